# Initial kernel scaffold; baseline (speedup 1.0000x reference)
import functools

import jax
import jax.numpy as jnp
from jax import lax
from jax.experimental import pallas as pl
from jax.experimental.pallas import tpu as pltpu
from jax.experimental.pallas import tpu_sc as plsc

N_USERS = 30000
N_ITEMS = 20000
N = N_USERS + N_ITEMS
D = 32
NNZ = 1600000
B = 4096
T_CONST = 2.0
GAMMA = 1.0

NPAD = 50176
NC, NS, L = 2, 16, 16
NW = NC * NS
EPW = NNZ // NW
CHUNK = 400
ZROWS = 784


def _spmm_kernel(adj_row, adj_col, adj_val, emb):
    mesh = plsc.VectorSubcoreMesh(core_axis_name="c", subcore_axis_name="s")

    @functools.partial(
        pl.kernel,
        mesh=mesh,
        out_type=(
            jax.ShapeDtypeStruct((NPAD, D), jnp.float32),
            jax.ShapeDtypeStruct((NPAD, D), jnp.float32),
        ),
        scratch_types=[
            pltpu.VMEM((CHUNK,), jnp.int32),
            pltpu.VMEM((CHUNK,), jnp.int32),
            pltpu.VMEM((CHUNK,), jnp.float32),
            pltpu.VMEM((CHUNK, D), jnp.float32),
            pltpu.VMEM((ZROWS, D), jnp.float32),
            pltpu.VMEM_SHARED((NPAD, D), jnp.float32),
            pltpu.SemaphoreType.DMA,
        ],
    )
    def k(rows_h, cols_h, vals_h, emb_h, out0, out1, colv, rowv, valv, gbuf,
          zbuf, accum, sem):
        cid = lax.axis_index("c")
        sid = lax.axis_index("s")
        wid = cid * NS + sid

        zeros16 = jnp.zeros((L,), jnp.float32)

        def zrow(i, _):
            zbuf[i, pl.ds(0, L)] = zeros16
            zbuf[i, pl.ds(L, L)] = zeros16
            return 0

        lax.fori_loop(0, ZROWS, zrow, 0)

        def zcopy(q, _):
            pltpu.sync_copy(zbuf, accum.at[pl.ds(sid * (NPAD // NS) + q * ZROWS, ZROWS)])
            return 0

        lax.fori_loop(0, (NPAD // NS) // ZROWS, zcopy, 0)
        plsc.subcore_barrier()

        base = wid * EPW

        def chunk(t, _):
            off = base + t * CHUNK
            pltpu.sync_copy(cols_h.at[pl.ds(off, CHUNK)], colv)
            pltpu.sync_copy(rows_h.at[pl.ds(off, CHUNK)], rowv)
            pltpu.sync_copy(vals_h.at[pl.ds(off, CHUNK)], valv)
            pltpu.async_copy(emb_h.at[colv], gbuf, sem).wait()

            def grp(g, _):
                v16 = valv[pl.ds(g * L, L)]
                for j in range(L):
                    e = g * L + j
                    vb = v16.at[jnp.full((L,), j, jnp.int32)].get(
                        mode="promise_in_bounds")
                    a = gbuf[e, pl.ds(0, L)] * vb
                    b = gbuf[e, pl.ds(L, L)] * vb
                    gbuf[e, pl.ds(0, L)] = a
                    gbuf[e, pl.ds(L, L)] = b
                return 0

            lax.fori_loop(0, CHUNK // L, grp, 0)
            pltpu.sync_copy(gbuf, accum.at[rowv], add=True)
            return 0

        lax.fori_loop(0, EPW // CHUNK, chunk, 0)
        plsc.subcore_barrier()

        def rd(q, _):
            r0 = sid * (NPAD // NS) + q * ZROWS
            pltpu.sync_copy(accum.at[pl.ds(r0, ZROWS)], zbuf)

            @pl.when(cid == 0)
            def _():
                pltpu.sync_copy(zbuf, out0.at[pl.ds(r0, ZROWS)])

            @pl.when(cid == 1)
            def _():
                pltpu.sync_copy(zbuf, out1.at[pl.ds(r0, ZROWS)])
            return 0

        lax.fori_loop(0, (NPAD // NS) // ZROWS, rd, 0)

    return k(adj_row, adj_col, adj_val, emb)


def _sc_gather(table, idx, nrows):
    mesh = plsc.VectorSubcoreMesh(core_axis_name="c", subcore_axis_name="s")
    per_w = nrows // NW

    @functools.partial(
        pl.kernel,
        mesh=mesh,
        out_type=jax.ShapeDtypeStruct((nrows, D), jnp.float32),
        scratch_types=[
            pltpu.VMEM((per_w,), jnp.int32),
            pltpu.VMEM((per_w, D), jnp.float32),
            pltpu.SemaphoreType.DMA,
        ],
    )
    def k(table_h, idx_h, out_h, idxv, buf, sem):
        wid = lax.axis_index("c") * NS + lax.axis_index("s")
        b0 = wid * per_w
        pltpu.sync_copy(idx_h.at[pl.ds(b0, per_w)], idxv)
        pltpu.async_copy(table_h.at[idxv], buf, sem).wait()
        pltpu.sync_copy(buf, out_h.at[pl.ds(b0, per_w)])

    return k(table, idx)


def _tc_add2(a, b):
    def body(a_ref, b_ref, o_ref):
        o_ref[...] = a_ref[...] + b_ref[...]

    blk = pl.BlockSpec((NPAD // 16, D), lambda i: (i, 0))
    return pl.pallas_call(
        body,
        grid=(16,),
        in_specs=[blk, blk],
        out_specs=blk,
        out_shape=jax.ShapeDtypeStruct((NPAD, D), jnp.float32),
    )(a, b)


def _tc_final_table(e1, p2a, p2b):
    def body(a_ref, b_ref, c_ref, o_ref):
        o_ref[...] = 0.5 * a_ref[...] + 0.5 * b_ref[...] + 0.5 * c_ref[...]

    blk = pl.BlockSpec((NPAD // 16, D), lambda i: (i, 0))
    return pl.pallas_call(
        body,
        grid=(16,),
        in_specs=[blk, blk, blk],
        out_specs=blk,
        out_shape=jax.ShapeDtypeStruct((NPAD, D), jnp.float32),
    )(e1, p2a, p2b)


def _tc_loss(rows, wu_c, wu_r, wp_c, wp_r):
    RB = 512

    def body(rows_ref, wuc_ref, wur_ref, wpc_ref, wpr_ref, o_ref, un_ref, pn_ref):
        def norm(x):
            return x / (jnp.sqrt(jnp.sum(x * x, axis=1, keepdims=True)) + 1e-12)

        ue = norm(rows_ref[pl.ds(0, B), :])
        ie = norm(rows_ref[pl.ds(B, B), :])
        un_ref[...] = norm(rows_ref[pl.ds(2 * B, B), :])
        pn_ref[...] = norm(rows_ref[pl.ds(3 * B, B), :])

        diff = ue - ie
        d = jnp.sqrt(jnp.sum(diff * diff, axis=1))
        t = d + 1e-12
        align = jnp.sum(t * t) / B

        def uniform(xn_ref, w_c, w_r):
            def blkstep(k, s):
                xb = xn_ref[pl.ds(k * RB, RB), :]
                g = lax.dot_general(xb, xn_ref[...],
                                    (((1,), (1,)), ((), ())),
                                    preferred_element_type=jnp.float32)
                sq = jnp.maximum(2.0 - 2.0 * g, 0.0)
                e = jnp.exp(-T_CONST * sq)
                wc = lax.dynamic_slice(w_c, (k * RB, 0), (RB, 1))
                return s + jnp.sum(e * w_r * wc)

            s = lax.fori_loop(0, B // RB, blkstep, 0.0)
            n = jnp.sum(w_r)
            return jnp.log((s - n) / (n * (n - 1.0)) + 1e-12)

        lu = uniform(un_ref, wuc_ref[...], wur_ref[...])
        lp = uniform(pn_ref, wpc_ref[...], wpr_ref[...])
        uni = GAMMA * (lu + lp) / 2.0

        out = jnp.zeros((1, 128), jnp.float32)
        out = out.at[0, 0].set(align).at[0, 1].set(uni)
        o_ref[...] = out

    return pl.pallas_call(
        body,
        out_shape=jax.ShapeDtypeStruct((1, 128), jnp.float32),
        scratch_shapes=[
            pltpu.VMEM((B, D), jnp.float32),
            pltpu.VMEM((B, D), jnp.float32),
        ],
    )(rows, wu_c, wu_r, wp_c, wp_r)


def kernel(user, positive, adj_row, adj_col, adj_val, user_table, item_table):
    user = user.astype(jnp.int32)
    positive = positive.astype(jnp.int32)
    adj_row = adj_row.astype(jnp.int32)
    adj_col = adj_col.astype(jnp.int32)

    emb0 = jnp.zeros((NPAD, D), jnp.float32)
    emb0 = emb0.at[:N_USERS].set(user_table).at[N_USERS:N].set(item_table)

    p1a, p1b = _spmm_kernel(adj_row, adj_col, adj_val, emb0)
    emb1 = _tc_add2(p1a, p1b)
    p2a, p2b = _spmm_kernel(adj_row, adj_col, adj_val, emb1)
    emb_f = _tc_final_table(emb1, p2a, p2b)

    su = jnp.sort(user)
    sp = jnp.sort(positive)
    cat_idx = jnp.concatenate([user, N_USERS + positive, su, N_USERS + sp])
    rows = _sc_gather(emb_f, cat_idx, 4 * B)

    wu = jnp.concatenate(
        [jnp.ones((1,), jnp.float32), (su[1:] != su[:-1]).astype(jnp.float32)])
    wp = jnp.concatenate(
        [jnp.ones((1,), jnp.float32), (sp[1:] != sp[:-1]).astype(jnp.float32)])
    o = _tc_loss(rows, wu.reshape(B, 1), wu.reshape(1, B),
                 wp.reshape(B, 1), wp.reshape(1, B))
    return jnp.stack([o[0, 0], o[0, 1]])


# speedup vs baseline: 11.4388x; 11.4388x over previous
import functools

import jax
import jax.numpy as jnp
from jax import lax
from jax.experimental import pallas as pl
from jax.experimental.pallas import tpu as pltpu
from jax.experimental.pallas import tpu_sc as plsc

N_USERS = 30000
N_ITEMS = 20000
N = N_USERS + N_ITEMS
D = 32
NNZ = 1600000
B = 4096
T_CONST = 2.0
GAMMA = 1.0

NPAD = 50176
NC, NS, L = 2, 16, 16
NW = NC * NS
EPW = NNZ // NW
SUB = 80
KSUB = 5
CHUNK = SUB * KSUB
ZROWS = 392


def _spmm_kernel(adj_row, adj_col, adj_val, emb):
    mesh = plsc.VectorSubcoreMesh(core_axis_name="c", subcore_axis_name="s")

    @functools.partial(
        pl.kernel,
        mesh=mesh,
        out_type=(
            jax.ShapeDtypeStruct((NPAD, D), jnp.float32),
            jax.ShapeDtypeStruct((NPAD, D), jnp.float32),
        ),
        scratch_types=[
            pltpu.VMEM((KSUB, SUB), jnp.int32),
            pltpu.VMEM((KSUB, SUB), jnp.int32),
            pltpu.VMEM((KSUB, SUB), jnp.float32),
            pltpu.VMEM((CHUNK, D), jnp.float32),
            pltpu.VMEM((ZROWS, D), jnp.float32),
            pltpu.VMEM_SHARED((NPAD, D), jnp.float32),
            pltpu.SemaphoreType.DMA,
        ],
        compiler_params=pltpu.CompilerParams(use_tc_tiling_on_sc=False),
    )
    def k(rows_h, cols_h, vals_h, emb_h, out0, out1, colv, rowv, valv, gbuf,
          zbuf, accum, sem):
        cid = lax.axis_index("c")
        sid = lax.axis_index("s")
        wid = cid * NS + sid

        zeros16 = jnp.zeros((L,), jnp.float32)

        def zrow(i, _):
            zbuf[i, pl.ds(0, L)] = zeros16
            zbuf[i, pl.ds(L, L)] = zeros16
            return 0

        lax.fori_loop(0, ZROWS, zrow, 0)

        def zcopy(q, _):
            pltpu.sync_copy(zbuf, accum.at[pl.ds(sid * (NPAD // NS) + q * ZROWS, ZROWS)])
            return 0

        lax.fori_loop(0, (NPAD // NS) // ZROWS, zcopy, 0)
        plsc.subcore_barrier()

        rbase = wid * (EPW // SUB)

        def chunk(t, _):
            r0 = rbase + t * KSUB
            pltpu.sync_copy(cols_h.at[pl.ds(r0, KSUB)], colv)
            pltpu.sync_copy(rows_h.at[pl.ds(r0, KSUB)], rowv)
            pltpu.sync_copy(vals_h.at[pl.ds(r0, KSUB)], valv)
            copies = [
                pltpu.async_copy(emb_h.at[colv.at[j]],
                                 gbuf.at[pl.ds(j * SUB, SUB)], sem)
                for j in range(KSUB)
            ]
            for cpy in copies:
                cpy.wait()

            for j in range(KSUB):
                def grp(g, _, j=j):
                    v16 = valv[j, pl.ds(g * L, L)]
                    for u in range(L):
                        e = j * SUB + g * L + u
                        vb = v16.at[jnp.full((L,), u, jnp.int32)].get(
                            mode="promise_in_bounds")
                        a = gbuf[e, pl.ds(0, L)] * vb
                        b = gbuf[e, pl.ds(L, L)] * vb
                        gbuf[e, pl.ds(0, L)] = a
                        gbuf[e, pl.ds(L, L)] = b
                    return 0

                lax.fori_loop(0, SUB // L, grp, 0)

            for j in range(KSUB):
                pltpu.sync_copy(gbuf.at[pl.ds(j * SUB, SUB)],
                                accum.at[rowv.at[j]], add=True)
            return 0

        lax.fori_loop(0, EPW // CHUNK, chunk, 0)
        plsc.subcore_barrier()

        def rd(q, _):
            r0 = sid * (NPAD // NS) + q * ZROWS
            pltpu.sync_copy(accum.at[pl.ds(r0, ZROWS)], zbuf)

            @pl.when(cid == 0)
            def _():
                pltpu.sync_copy(zbuf, out0.at[pl.ds(r0, ZROWS)])

            @pl.when(cid == 1)
            def _():
                pltpu.sync_copy(zbuf, out1.at[pl.ds(r0, ZROWS)])
            return 0

        lax.fori_loop(0, (NPAD // NS) // ZROWS, rd, 0)

    nr = NNZ // SUB
    return k(adj_row.reshape(nr, SUB), adj_col.reshape(nr, SUB),
             adj_val.reshape(nr, SUB), emb)


def _sc_gather(table, idx, nrows):
    mesh = plsc.VectorSubcoreMesh(core_axis_name="c", subcore_axis_name="s")
    per_w = nrows // NW
    kq = per_w // 128

    @functools.partial(
        pl.kernel,
        mesh=mesh,
        out_type=jax.ShapeDtypeStruct((nrows, D), jnp.float32),
        scratch_types=[
            pltpu.VMEM((kq, 128), jnp.int32),
            pltpu.VMEM((per_w, D), jnp.float32),
            pltpu.SemaphoreType.DMA,
        ],
        compiler_params=pltpu.CompilerParams(use_tc_tiling_on_sc=False),
    )
    def k(table_h, idx_h, out_h, idxv, buf, sem):
        wid = lax.axis_index("c") * NS + lax.axis_index("s")
        pltpu.sync_copy(idx_h.at[pl.ds(wid * kq, kq)], idxv)
        copies = [
            pltpu.async_copy(table_h.at[idxv.at[q]],
                             buf.at[pl.ds(q * 128, 128)], sem)
            for q in range(kq)
        ]
        for cpy in copies:
            cpy.wait()
        pltpu.sync_copy(buf, out_h.at[pl.ds(wid * per_w, per_w)])

    return k(table, idx.reshape(nrows // 128, 128))


def _tc_add2(a, b):
    def body(a_ref, b_ref, o_ref):
        o_ref[...] = a_ref[...] + b_ref[...]

    blk = pl.BlockSpec((NPAD // 16, D), lambda i: (i, 0))
    return pl.pallas_call(
        body,
        grid=(16,),
        in_specs=[blk, blk],
        out_specs=blk,
        out_shape=jax.ShapeDtypeStruct((NPAD, D), jnp.float32),
    )(a, b)


def _tc_final_table(e1, p2a, p2b):
    def body(a_ref, b_ref, c_ref, o_ref):
        o_ref[...] = 0.5 * a_ref[...] + 0.5 * b_ref[...] + 0.5 * c_ref[...]

    blk = pl.BlockSpec((NPAD // 16, D), lambda i: (i, 0))
    return pl.pallas_call(
        body,
        grid=(16,),
        in_specs=[blk, blk, blk],
        out_specs=blk,
        out_shape=jax.ShapeDtypeStruct((NPAD, D), jnp.float32),
    )(e1, p2a, p2b)


def _tc_loss(rows, wu_c, wu_r, wp_c, wp_r):
    RB = 512

    def body(rows_ref, wuc_ref, wur_ref, wpc_ref, wpr_ref, o_ref, un_ref, pn_ref):
        def norm(x):
            return x / (jnp.sqrt(jnp.sum(x * x, axis=1, keepdims=True)) + 1e-12)

        ue = norm(rows_ref[pl.ds(0, B), :])
        ie = norm(rows_ref[pl.ds(B, B), :])
        un_ref[...] = norm(rows_ref[pl.ds(2 * B, B), :])
        pn_ref[...] = norm(rows_ref[pl.ds(3 * B, B), :])

        diff = ue - ie
        d = jnp.sqrt(jnp.sum(diff * diff, axis=1))
        t = d + 1e-12
        align = jnp.sum(t * t) / B

        def uniform(xn_ref, wc_ref, w_r):
            def blkstep(k, s):
                xb = xn_ref[pl.ds(k * RB, RB), :]
                g = lax.dot_general(xb, xn_ref[...],
                                    (((1,), (1,)), ((), ())),
                                    preferred_element_type=jnp.float32)
                sq = jnp.maximum(2.0 - 2.0 * g, 0.0)
                e = jnp.exp(-T_CONST * sq)
                wc = wc_ref[pl.ds(k * RB, RB), :]
                return s + jnp.sum(e * w_r * wc)

            s = lax.fori_loop(0, B // RB, blkstep, 0.0)
            n = jnp.sum(w_r)
            return jnp.log((s - n) / (n * (n - 1.0)) + 1e-12)

        lu = uniform(un_ref, wuc_ref, wur_ref[...])
        lp = uniform(pn_ref, wpc_ref, wpr_ref[...])
        uni = GAMMA * (lu + lp) / 2.0

        li = lax.broadcasted_iota(jnp.int32, (1, 128), 1)
        o_ref[...] = jnp.where(li == 0, align,
                               jnp.where(li == 1, uni, 0.0))

    return pl.pallas_call(
        body,
        out_shape=jax.ShapeDtypeStruct((1, 128), jnp.float32),
        scratch_shapes=[
            pltpu.VMEM((B, D), jnp.float32),
            pltpu.VMEM((B, D), jnp.float32),
        ],
    )(rows, wu_c, wu_r, wp_c, wp_r)


def kernel(user, positive, adj_row, adj_col, adj_val, user_table, item_table):
    user = user.astype(jnp.int32)
    positive = positive.astype(jnp.int32)
    adj_row = adj_row.astype(jnp.int32)
    adj_col = adj_col.astype(jnp.int32)

    emb0 = jnp.zeros((NPAD, D), jnp.float32)
    emb0 = emb0.at[:N_USERS].set(user_table).at[N_USERS:N].set(item_table)

    p1a, p1b = _spmm_kernel(adj_row, adj_col, adj_val, emb0)
    emb1 = _tc_add2(p1a, p1b)
    p2a, p2b = _spmm_kernel(adj_row, adj_col, adj_val, emb1)
    emb_f = _tc_final_table(emb1, p2a, p2b)

    su = jnp.sort(user)
    sp = jnp.sort(positive)
    cat_idx = jnp.concatenate([user, N_USERS + positive, su, N_USERS + sp])
    rows = _sc_gather(emb_f, cat_idx, 4 * B)

    wu = jnp.concatenate(
        [jnp.ones((1,), jnp.float32), (su[1:] != su[:-1]).astype(jnp.float32)])
    wp = jnp.concatenate(
        [jnp.ones((1,), jnp.float32), (sp[1:] != sp[:-1]).astype(jnp.float32)])
    o = _tc_loss(rows, wu.reshape(B, 1), wu.reshape(1, B),
                 wp.reshape(B, 1), wp.reshape(1, B))
    return jnp.stack([o[0, 0], o[0, 1]])


# speedup vs baseline: 18.1714x; 1.5886x over previous
import functools

import jax
import jax.numpy as jnp
from jax import lax
from jax.experimental import pallas as pl
from jax.experimental.pallas import tpu as pltpu
from jax.experimental.pallas import tpu_sc as plsc

N_USERS = 30000
N_ITEMS = 20000
N = N_USERS + N_ITEMS
D = 32
NNZ = 1600000
B = 4096
T_CONST = 2.0
GAMMA = 1.0

NPAD = 50048
NC, NS, L = 2, 16, 16
NW = NC * NS
EPW = NNZ // NW
SUB = 80
KSUB = 5
CHUNK = SUB * KSUB
NCH = EPW // CHUNK
ZROWS = 391


def _spmm_kernel(adj_row, adj_col, adj_val, emb):
    mesh = plsc.VectorSubcoreMesh(core_axis_name="c", subcore_axis_name="s")

    @functools.partial(
        pl.kernel,
        mesh=mesh,
        out_type=(
            jax.ShapeDtypeStruct((NPAD, D), jnp.float32),
            jax.ShapeDtypeStruct((NPAD, D), jnp.float32),
        ),
        scratch_types=[
            pltpu.VMEM((2, KSUB, SUB), jnp.int32),
            pltpu.VMEM((2, KSUB, SUB), jnp.int32),
            pltpu.VMEM((2, KSUB, SUB), jnp.float32),
            pltpu.VMEM((CHUNK, D), jnp.float32),
            pltpu.VMEM((CHUNK, D), jnp.float32),
            pltpu.VMEM_SHARED((NPAD, D), jnp.float32),
            pltpu.SemaphoreType.DMA,
            pltpu.SemaphoreType.DMA,
            pltpu.SemaphoreType.DMA,
            pltpu.SemaphoreType.DMA,
        ],
        compiler_params=pltpu.CompilerParams(use_tc_tiling_on_sc=False),
    )
    def k(rows_h, cols_h, vals_h, emb_h, out0, out1, colv, rowv, valv, gbuf0,
          gbuf1, accum, sl0, sl1, sg0, sg1):
        cid = lax.axis_index("c")
        sid = lax.axis_index("s")
        wid = cid * NS + sid
        gbufs = (gbuf0, gbuf1)
        sls = (sl0, sl1)
        sgs = (sg0, sg1)

        zeros16 = jnp.zeros((L,), jnp.float32)

        def zrow(i, _):
            gbuf0[i, pl.ds(0, L)] = zeros16
            gbuf0[i, pl.ds(L, L)] = zeros16
            return 0

        lax.fori_loop(0, ZROWS, zrow, 0)

        def zcopy(q, _):
            pltpu.sync_copy(gbuf0.at[pl.ds(0, ZROWS)],
                            accum.at[pl.ds(sid * (NPAD // NS) + q * ZROWS, ZROWS)])
            return 0

        lax.fori_loop(0, (NPAD // NS) // ZROWS, zcopy, 0)
        plsc.subcore_barrier()

        rbase = wid * (EPW // SUB)

        def fire_loads(c, b):
            r0 = rbase + c * KSUB
            pltpu.async_copy(cols_h.at[pl.ds(r0, KSUB)], colv.at[b], sls[b])
            pltpu.async_copy(rows_h.at[pl.ds(r0, KSUB)], rowv.at[b], sls[b])
            pltpu.async_copy(vals_h.at[pl.ds(r0, KSUB)], valv.at[b], sls[b])

        def drain_loads(b):
            pltpu.make_async_copy(cols_h.at[pl.ds(0, KSUB)], colv.at[b],
                                  sls[b]).wait()
            pltpu.make_async_copy(rows_h.at[pl.ds(0, KSUB)], rowv.at[b],
                                  sls[b]).wait()
            pltpu.make_async_copy(vals_h.at[pl.ds(0, KSUB)], valv.at[b],
                                  sls[b]).wait()

        def fire_gathers(b):
            for j in range(KSUB):
                pltpu.async_copy(emb_h.at[colv.at[b, j]],
                                 gbufs[b].at[pl.ds(j * SUB, SUB)], sgs[b])

        def drain_gathers(b):
            for j in range(KSUB):
                pltpu.make_async_copy(emb_h.at[pl.ds(0, SUB)],
                                      gbufs[b].at[pl.ds(j * SUB, SUB)],
                                      sgs[b]).wait()

        def scale(b):
            g_ref = gbufs[b]
            for j in range(KSUB):
                def grp(g, _, j=j):
                    v16 = valv[b, j, pl.ds(g * L, L)]
                    for u in range(L):
                        e = j * SUB + g * L + u
                        vb = v16.at[jnp.full((L,), u, jnp.int32)].get(
                            mode="promise_in_bounds")
                        a = g_ref[e, pl.ds(0, L)] * vb
                        bb = g_ref[e, pl.ds(L, L)] * vb
                        g_ref[e, pl.ds(0, L)] = a
                        g_ref[e, pl.ds(L, L)] = bb
                    return 0

                lax.fori_loop(0, SUB // L, grp, 0)

        def scatter(b):
            for j in range(KSUB):
                pltpu.sync_copy(gbufs[b].at[pl.ds(j * SUB, SUB)],
                                accum.at[rowv.at[b, j]], add=True)

        fire_loads(0, 0)
        drain_loads(0)
        fire_gathers(0)
        fire_loads(1, 1)

        def pair(p, _):
            for b in (0, 1):
                c = 2 * p + b
                drain_gathers(b)
                drain_loads(1 - b)
                fire_gathers(1 - b)
                scale(b)
                scatter(b)
                r_next = jnp.minimum(c + 2, NCH - 1)
                fire_loads(r_next, b)
            return 0

        lax.fori_loop(0, (NCH - 1) // 2, pair, 0)

        drain_gathers(0)
        scale(0)
        scatter(0)
        drain_loads(1)
        plsc.subcore_barrier()

        def rd(q, _):
            r0 = sid * (NPAD // NS) + q * ZROWS
            pltpu.sync_copy(accum.at[pl.ds(r0, ZROWS)], gbuf0.at[pl.ds(0, ZROWS)])

            @pl.when(cid == 0)
            def _():
                pltpu.sync_copy(gbuf0.at[pl.ds(0, ZROWS)], out0.at[pl.ds(r0, ZROWS)])

            @pl.when(cid == 1)
            def _():
                pltpu.sync_copy(gbuf0.at[pl.ds(0, ZROWS)], out1.at[pl.ds(r0, ZROWS)])
            return 0

        lax.fori_loop(0, (NPAD // NS) // ZROWS, rd, 0)

    nr = NNZ // SUB
    return k(adj_row.reshape(nr, SUB), adj_col.reshape(nr, SUB),
             adj_val.reshape(nr, SUB), emb)


def _sc_gather(table, idx, nrows):
    mesh = plsc.VectorSubcoreMesh(core_axis_name="c", subcore_axis_name="s")
    per_w = nrows // NW
    kq = per_w // 128

    @functools.partial(
        pl.kernel,
        mesh=mesh,
        out_type=jax.ShapeDtypeStruct((nrows, D), jnp.float32),
        scratch_types=[
            pltpu.VMEM((kq, 128), jnp.int32),
            pltpu.VMEM((per_w, D), jnp.float32),
            pltpu.SemaphoreType.DMA,
        ],
        compiler_params=pltpu.CompilerParams(use_tc_tiling_on_sc=False),
    )
    def k(table_h, idx_h, out_h, idxv, buf, sem):
        wid = lax.axis_index("c") * NS + lax.axis_index("s")
        pltpu.sync_copy(idx_h.at[pl.ds(wid * kq, kq)], idxv)
        copies = [
            pltpu.async_copy(table_h.at[idxv.at[q]],
                             buf.at[pl.ds(q * 128, 128)], sem)
            for q in range(kq)
        ]
        for cpy in copies:
            cpy.wait()
        pltpu.sync_copy(buf, out_h.at[pl.ds(wid * per_w, per_w)])

    return k(table, idx.reshape(nrows // 128, 128))


def _tc_add2(a, b):
    def body(a_ref, b_ref, o_ref):
        o_ref[...] = a_ref[...] + b_ref[...]

    blk = pl.BlockSpec((NPAD // 16, D), lambda i: (i, 0))
    return pl.pallas_call(
        body,
        grid=(16,),
        in_specs=[blk, blk],
        out_specs=blk,
        out_shape=jax.ShapeDtypeStruct((NPAD, D), jnp.float32),
    )(a, b)


def _tc_final_table(e1, p2a, p2b):
    def body(a_ref, b_ref, c_ref, o_ref):
        o_ref[...] = 0.5 * a_ref[...] + 0.5 * b_ref[...] + 0.5 * c_ref[...]

    blk = pl.BlockSpec((NPAD // 16, D), lambda i: (i, 0))
    return pl.pallas_call(
        body,
        grid=(16,),
        in_specs=[blk, blk, blk],
        out_specs=blk,
        out_shape=jax.ShapeDtypeStruct((NPAD, D), jnp.float32),
    )(e1, p2a, p2b)


def _tc_loss(rows, wu_c, wu_r, wp_c, wp_r):
    RB = 512

    def body(rows_ref, wuc_ref, wur_ref, wpc_ref, wpr_ref, o_ref, un_ref, pn_ref):
        def norm(x):
            return x / (jnp.sqrt(jnp.sum(x * x, axis=1, keepdims=True)) + 1e-12)

        ue = norm(rows_ref[pl.ds(0, B), :])
        ie = norm(rows_ref[pl.ds(B, B), :])
        un_ref[...] = norm(rows_ref[pl.ds(2 * B, B), :])
        pn_ref[...] = norm(rows_ref[pl.ds(3 * B, B), :])

        diff = ue - ie
        d = jnp.sqrt(jnp.sum(diff * diff, axis=1))
        t = d + 1e-12
        align = jnp.sum(t * t) / B

        def uniform(xn_ref, wc_ref, w_r):
            def blkstep(k, s):
                xb = xn_ref[pl.ds(k * RB, RB), :]
                g = lax.dot_general(xb, xn_ref[...],
                                    (((1,), (1,)), ((), ())),
                                    preferred_element_type=jnp.float32)
                sq = jnp.maximum(2.0 - 2.0 * g, 0.0)
                e = jnp.exp(-T_CONST * sq)
                wc = wc_ref[pl.ds(k * RB, RB), :]
                return s + jnp.sum(e * w_r * wc)

            s = lax.fori_loop(0, B // RB, blkstep, 0.0)
            n = jnp.sum(w_r)
            return jnp.log((s - n) / (n * (n - 1.0)) + 1e-12)

        lu = uniform(un_ref, wuc_ref, wur_ref[...])
        lp = uniform(pn_ref, wpc_ref, wpr_ref[...])
        uni = GAMMA * (lu + lp) / 2.0

        li = lax.broadcasted_iota(jnp.int32, (1, 128), 1)
        o_ref[...] = jnp.where(li == 0, align,
                               jnp.where(li == 1, uni, 0.0))

    return pl.pallas_call(
        body,
        out_shape=jax.ShapeDtypeStruct((1, 128), jnp.float32),
        scratch_shapes=[
            pltpu.VMEM((B, D), jnp.float32),
            pltpu.VMEM((B, D), jnp.float32),
        ],
    )(rows, wu_c, wu_r, wp_c, wp_r)


def kernel(user, positive, adj_row, adj_col, adj_val, user_table, item_table):
    user = user.astype(jnp.int32)
    positive = positive.astype(jnp.int32)
    adj_row = adj_row.astype(jnp.int32)
    adj_col = adj_col.astype(jnp.int32)

    emb0 = jnp.zeros((NPAD, D), jnp.float32)
    emb0 = emb0.at[:N_USERS].set(user_table).at[N_USERS:N].set(item_table)

    p1a, p1b = _spmm_kernel(adj_row, adj_col, adj_val, emb0)
    emb1 = _tc_add2(p1a, p1b)
    p2a, p2b = _spmm_kernel(adj_row, adj_col, adj_val, emb1)
    emb_f = _tc_final_table(emb1, p2a, p2b)

    su = jnp.sort(user)
    sp = jnp.sort(positive)
    cat_idx = jnp.concatenate([user, N_USERS + positive, su, N_USERS + sp])
    rows = _sc_gather(emb_f, cat_idx, 4 * B)

    wu = jnp.concatenate(
        [jnp.ones((1,), jnp.float32), (su[1:] != su[:-1]).astype(jnp.float32)])
    wp = jnp.concatenate(
        [jnp.ones((1,), jnp.float32), (sp[1:] != sp[:-1]).astype(jnp.float32)])
    o = _tc_loss(rows, wu.reshape(B, 1), wu.reshape(1, B),
                 wp.reshape(B, 1), wp.reshape(1, B))
    return jnp.stack([o[0, 0], o[0, 1]])


# speedup vs baseline: 26.4968x; 1.4582x over previous
import functools

import jax
import jax.numpy as jnp
from jax import lax
from jax.experimental import pallas as pl
from jax.experimental.pallas import tpu as pltpu
from jax.experimental.pallas import tpu_sc as plsc

N_USERS = 30000
N_ITEMS = 20000
N = N_USERS + N_ITEMS
D = 32
NNZ = 1600000
B = 4096
T_CONST = 2.0
GAMMA = 1.0

NPAD = 50048
NC, NS, L = 2, 16, 16
NW = NC * NS
EPW = NNZ // NW
SUB = 80
KSUB = 5
CHUNK = SUB * KSUB
NCH = EPW // CHUNK
ZROWS = 391


def _spmm_kernel(adj_row, adj_col, adj_val, emb):
    mesh = plsc.VectorSubcoreMesh(core_axis_name="c", subcore_axis_name="s")

    @functools.partial(
        pl.kernel,
        mesh=mesh,
        out_type=(
            jax.ShapeDtypeStruct((NPAD, D), jnp.float32),
            jax.ShapeDtypeStruct((NPAD, D), jnp.float32),
        ),
        scratch_types=[
            pltpu.VMEM((2, KSUB, SUB), jnp.int32),
            pltpu.VMEM((2, KSUB, SUB), jnp.int32),
            pltpu.VMEM((2, KSUB, SUB), jnp.float32),
            pltpu.VMEM((2, KSUB, SUB), jnp.int32),
            pltpu.VMEM((CHUNK, D), jnp.float32),
            pltpu.VMEM((CHUNK, D), jnp.float32),
            pltpu.VMEM_SHARED((NPAD, D), jnp.float32),
            pltpu.SemaphoreType.DMA,
            pltpu.SemaphoreType.DMA,
            pltpu.SemaphoreType.DMA,
            pltpu.SemaphoreType.DMA,
            pltpu.SemaphoreType.DMA,
            pltpu.SemaphoreType.DMA,
        ],
        compiler_params=pltpu.CompilerParams(use_tc_tiling_on_sc=False),
    )
    def k(rows_h, cols_h, vals_h, emb_h, out0, out1, colv, rowv, valv, rsc,
          gbuf0, gbuf1, accum, sl0, sl1, sg0, sg1, ss0, ss1):
        cid = lax.axis_index("c")
        sid = lax.axis_index("s")
        wid = cid * NS + sid
        gbufs = (gbuf0, gbuf1)
        sls = (sl0, sl1)
        sgs = (sg0, sg1)
        sss = (ss0, ss1)

        zeros16 = jnp.zeros((L,), jnp.float32)

        def zrow(i, _):
            gbuf0[i, pl.ds(0, L)] = zeros16
            gbuf0[i, pl.ds(L, L)] = zeros16
            return 0

        lax.fori_loop(0, ZROWS, zrow, 0)

        def zcopy(q, _):
            pltpu.sync_copy(gbuf0.at[pl.ds(0, ZROWS)],
                            accum.at[pl.ds(sid * (NPAD // NS) + q * ZROWS, ZROWS)])
            return 0

        lax.fori_loop(0, (NPAD // NS) // ZROWS, zcopy, 0)
        plsc.subcore_barrier()

        rbase = wid * (EPW // SUB)

        def fire_loads(c, b):
            r0 = rbase + c * KSUB
            pltpu.async_copy(cols_h.at[pl.ds(r0, KSUB)], colv.at[b], sls[b])
            pltpu.async_copy(rows_h.at[pl.ds(r0, KSUB)], rowv.at[b], sls[b])
            pltpu.async_copy(vals_h.at[pl.ds(r0, KSUB)], valv.at[b], sls[b])

        def drain_loads(b):
            pltpu.make_async_copy(cols_h.at[pl.ds(0, KSUB)], colv.at[b],
                                  sls[b]).wait()
            pltpu.make_async_copy(rows_h.at[pl.ds(0, KSUB)], rowv.at[b],
                                  sls[b]).wait()
            pltpu.make_async_copy(vals_h.at[pl.ds(0, KSUB)], valv.at[b],
                                  sls[b]).wait()

        def fire_gathers(b):
            for j in range(KSUB):
                pltpu.async_copy(emb_h.at[colv.at[b, j]],
                                 gbufs[b].at[pl.ds(j * SUB, SUB)], sgs[b])

        def drain_gathers(b):
            for j in range(KSUB):
                pltpu.make_async_copy(emb_h.at[pl.ds(0, SUB)],
                                      gbufs[b].at[pl.ds(j * SUB, SUB)],
                                      sgs[b]).wait()

        def scale(b):
            g_ref = gbufs[b]
            for j in range(KSUB):
                def grp(g, _, j=j):
                    v16 = valv[b, j, pl.ds(g * L, L)]
                    for u in range(L):
                        e = j * SUB + g * L + u
                        vb = v16.at[jnp.full((L,), u, jnp.int32)].get(
                            mode="promise_in_bounds")
                        a = g_ref[e, pl.ds(0, L)] * vb
                        bb = g_ref[e, pl.ds(L, L)] * vb
                        g_ref[e, pl.ds(0, L)] = a
                        g_ref[e, pl.ds(L, L)] = bb
                    return 0

                lax.fori_loop(0, SUB // L, grp, 0)

        def copy_scatter_idx(b):
            for j in range(KSUB):
                def cp(g, _, j=j):
                    rsc[b, j, pl.ds(g * L, L)] = rowv[b, j, pl.ds(g * L, L)]
                    return 0

                lax.fori_loop(0, SUB // L, cp, 0)

        def fire_scatter(b):
            for j in range(KSUB):
                pltpu.async_copy(gbufs[b].at[pl.ds(j * SUB, SUB)],
                                 accum.at[rsc.at[b, j]], sss[b], add=True)

        def drain_scatter(b):
            for j in range(KSUB):
                pltpu.make_async_copy(gbufs[b].at[pl.ds(j * SUB, SUB)],
                                      accum.at[pl.ds(0, SUB)], sss[b]).wait()

        def step(c, b, first, last):
            drain_gathers(b)
            drain_loads(1 - b)
            if not first:
                drain_scatter(1 - b)
            fire_gathers(1 - b)
            copy_scatter_idx(b)
            scale(b)
            fire_scatter(b)
            r_next = jnp.minimum(c + 2, NCH - 1)
            fire_loads(r_next, b)

        fire_loads(0, 0)
        drain_loads(0)
        fire_gathers(0)
        fire_loads(1, 1)
        step(0, 0, first=True, last=False)
        step(1, 1, first=False, last=False)

        def pair(p, _):
            for b in (0, 1):
                step(2 * p + 2 + b, b, first=False, last=False)
            return 0

        lax.fori_loop(0, (NCH - 3) // 2, pair, 0)

        drain_gathers(0)
        drain_scatter(1)
        copy_scatter_idx(0)
        scale(0)
        fire_scatter(0)
        drain_scatter(0)
        drain_loads(1)
        plsc.subcore_barrier()

        def rd(q, _):
            r0 = sid * (NPAD // NS) + q * ZROWS
            pltpu.sync_copy(accum.at[pl.ds(r0, ZROWS)], gbuf0.at[pl.ds(0, ZROWS)])

            @pl.when(cid == 0)
            def _():
                pltpu.sync_copy(gbuf0.at[pl.ds(0, ZROWS)], out0.at[pl.ds(r0, ZROWS)])

            @pl.when(cid == 1)
            def _():
                pltpu.sync_copy(gbuf0.at[pl.ds(0, ZROWS)], out1.at[pl.ds(r0, ZROWS)])
            return 0

        lax.fori_loop(0, (NPAD // NS) // ZROWS, rd, 0)

    nr = NNZ // SUB
    return k(adj_row.reshape(nr, SUB), adj_col.reshape(nr, SUB),
             adj_val.reshape(nr, SUB), emb)


def _sc_gather3(t0, t1, t2, idx, nrows):
    mesh = plsc.VectorSubcoreMesh(core_axis_name="c", subcore_axis_name="s")
    per_w = nrows // NW
    kq = per_w // 128

    @functools.partial(
        pl.kernel,
        mesh=mesh,
        out_type=jax.ShapeDtypeStruct((nrows, D), jnp.float32),
        scratch_types=[
            pltpu.VMEM((kq, 128), jnp.int32),
            pltpu.VMEM((per_w, D), jnp.float32),
            pltpu.VMEM((per_w, D), jnp.float32),
            pltpu.VMEM((per_w, D), jnp.float32),
            pltpu.SemaphoreType.DMA,
        ],
        compiler_params=pltpu.CompilerParams(use_tc_tiling_on_sc=False),
    )
    def k(t0_h, t1_h, t2_h, idx_h, out_h, idxv, b0, b1, b2, sem):
        wid = lax.axis_index("c") * NS + lax.axis_index("s")
        pltpu.sync_copy(idx_h.at[pl.ds(wid * kq, kq)], idxv)
        copies = []
        for tab, buf in ((t0_h, b0), (t1_h, b1), (t2_h, b2)):
            for q in range(kq):
                copies.append(
                    pltpu.async_copy(tab.at[idxv.at[q]],
                                     buf.at[pl.ds(q * 128, 128)], sem))
        for cpy in copies:
            cpy.wait()

        def addrow(r, _):
            a0 = b0[r, pl.ds(0, L)] + b1[r, pl.ds(0, L)] + b2[r, pl.ds(0, L)]
            a1 = b0[r, pl.ds(L, L)] + b1[r, pl.ds(L, L)] + b2[r, pl.ds(L, L)]
            b0[r, pl.ds(0, L)] = a0
            b0[r, pl.ds(L, L)] = a1
            return 0

        lax.fori_loop(0, per_w, addrow, 0)
        pltpu.sync_copy(b0, out_h.at[pl.ds(wid * per_w, per_w)])

    return k(t0, t1, t2, idx.reshape(nrows // 128, 128))


def _tc_add2(a, b):
    def body(a_ref, b_ref, o_ref):
        o_ref[...] = a_ref[...] + b_ref[...]

    blk = pl.BlockSpec((NPAD // 16, D), lambda i: (i, 0))
    return pl.pallas_call(
        body,
        grid=(16,),
        in_specs=[blk, blk],
        out_specs=blk,
        out_shape=jax.ShapeDtypeStruct((NPAD, D), jnp.float32),
    )(a, b)


def _tc_loss(rows, wu_c, wp_c):
    RB = 512

    def body(rows_ref, wuc_ref, wpc_ref, o_ref, un_ref, pn_ref):
        def norm(x):
            return x / (jnp.sqrt(jnp.sum(x * x, axis=1, keepdims=True)) + 1e-12)

        ue = norm(rows_ref[pl.ds(0, B), :])
        ie = norm(rows_ref[pl.ds(B, B), :])
        un_ref[...] = norm(rows_ref[pl.ds(2 * B, B), :]).astype(jnp.bfloat16)
        pn_ref[...] = norm(rows_ref[pl.ds(3 * B, B), :]).astype(jnp.bfloat16)

        diff = ue - ie
        d = jnp.sqrt(jnp.sum(diff * diff, axis=1))
        t = d + 1e-12
        align = jnp.sum(t * t) / B

        def uniform(xn_ref, wc_ref):
            w_full = wc_ref[...]

            def blkstep(k, s):
                xb = xn_ref[pl.ds(k * RB, RB), :]
                g = lax.dot_general(xb, xn_ref[...],
                                    (((1,), (1,)), ((), ())),
                                    preferred_element_type=jnp.float32)
                sq = jnp.maximum(2.0 - 2.0 * g, 0.0)
                e = jnp.exp(-T_CONST * sq)
                ew = lax.dot_general(e, w_full, (((1,), (0,)), ((), ())),
                                     preferred_element_type=jnp.float32)
                wc = wc_ref[pl.ds(k * RB, RB), :]
                return s + jnp.sum(ew * wc)

            s = lax.fori_loop(0, B // RB, blkstep, 0.0)
            n = jnp.sum(w_full)
            return jnp.log((s - n) / (n * (n - 1.0)) + 1e-12)

        lu = uniform(un_ref, wuc_ref)
        lp = uniform(pn_ref, wpc_ref)
        uni = GAMMA * (lu + lp) / 2.0

        li = lax.broadcasted_iota(jnp.int32, (1, 128), 1)
        o_ref[...] = jnp.where(li == 0, align,
                               jnp.where(li == 1, uni, 0.0))

    return pl.pallas_call(
        body,
        out_shape=jax.ShapeDtypeStruct((1, 128), jnp.float32),
        scratch_shapes=[
            pltpu.VMEM((B, D), jnp.bfloat16),
            pltpu.VMEM((B, D), jnp.bfloat16),
        ],
    )(rows, wu_c, wp_c)


def kernel(user, positive, adj_row, adj_col, adj_val, user_table, item_table):
    user = user.astype(jnp.int32)
    positive = positive.astype(jnp.int32)
    adj_row = adj_row.astype(jnp.int32)
    adj_col = adj_col.astype(jnp.int32)

    emb0 = jnp.zeros((NPAD, D), jnp.float32)
    emb0 = emb0.at[:N_USERS].set(user_table).at[N_USERS:N].set(item_table)

    p1a, p1b = _spmm_kernel(adj_row, adj_col, adj_val, emb0)
    emb1 = _tc_add2(p1a, p1b)
    p2a, p2b = _spmm_kernel(adj_row, adj_col, adj_val, emb1)

    su = jnp.sort(user)
    sp = jnp.sort(positive)
    cat_idx = jnp.concatenate([user, N_USERS + positive, su, N_USERS + sp])
    rows = _sc_gather3(emb1, p2a, p2b, cat_idx, 4 * B)

    wu = jnp.concatenate(
        [jnp.ones((1,), jnp.float32), (su[1:] != su[:-1]).astype(jnp.float32)])
    wp = jnp.concatenate(
        [jnp.ones((1,), jnp.float32), (sp[1:] != sp[:-1]).astype(jnp.float32)])
    o = _tc_loss(rows, wu.reshape(B, 1), wp.reshape(B, 1))
    return jnp.stack([o[0, 0], o[0, 1]])
